# Initial kernel scaffold; baseline (speedup 1.0000x reference)
import jax
import jax.numpy as jnp
from jax.experimental import pallas as pl


def kernel(x, edge_index, batch, W1, b1, W2, b2, Wout, bout):
    raise NotImplementedError("write your pallas kernel here")


# speedup vs baseline: 5.7289x; 5.7289x over previous
import functools

import jax
import jax.numpy as jnp
from jax import lax
from jax.experimental import pallas as pl
from jax.experimental.pallas import tpu as pltpu
from jax.experimental.pallas import tpu_sc as plsc

N = 10000
E = 320000
D = 128
G = 8
NP = 10240
CH = 128
NTILES = 32
CHUNKS = -(-E // (NTILES * CH))
EPT = CHUNKS * CH
EP = EPT * NTILES
RPT = NP // 16
BR = 1280
GRID = NP // BR

_mesh = plsc.VectorSubcoreMesh(core_axis_name="c", subcore_axis_name="s")


@functools.partial(
    pl.kernel, mesh=_mesh,
    out_type=jax.ShapeDtypeStruct((2, NP, D), jnp.float32),
    scratch_types=[
        pltpu.VMEM_SHARED((NP, D), jnp.float32),
        pltpu.VMEM((CH,), jnp.int32),
        pltpu.VMEM((CH, D), jnp.float32),
    ],
)
def _sc_degree(dst_hbm, deg_hbm, accd, didx, ones_v):
    cid = lax.axis_index("c")
    sid = lax.axis_index("s")

    def fill(i, carry):
        for j in range(D // 16):
            ones_v[i, pl.ds(j * 16, 16)] = jnp.zeros((16,), jnp.float32)
        return carry
    lax.fori_loop(0, CH, fill, 0)

    for j in range(RPT // CH):
        pltpu.sync_copy(ones_v, accd.at[pl.ds(sid * RPT + j * CH, CH)])

    def fill2(i, carry):
        for j in range(D // 16):
            ones_v[i, pl.ds(j * 16, 16)] = jnp.ones((16,), jnp.float32)
        return carry
    lax.fori_loop(0, CH, fill2, 0)
    plsc.subcore_barrier()

    base = (cid * 16 + sid) * EPT

    def body(c, carry):
        pltpu.sync_copy(dst_hbm.at[pl.ds(base + c * CH, CH)], didx)
        pltpu.sync_copy(ones_v, accd.at[didx], add=True)
        return carry
    lax.fori_loop(0, CHUNKS, body, 0)
    plsc.subcore_barrier()

    pltpu.sync_copy(accd.at[pl.ds(sid * RPT, RPT)],
                    deg_hbm.at[cid, pl.ds(sid * RPT, RPT)])


@functools.partial(
    pl.kernel, mesh=_mesh,
    out_type=jax.ShapeDtypeStruct((2, NP, D), jnp.float32),
    scratch_types=[
        pltpu.VMEM_SHARED((NP, D), jnp.float32),
        pltpu.VMEM((CH,), jnp.int32),
        pltpu.VMEM((CH,), jnp.int32),
        pltpu.VMEM((CH, D), jnp.float32),
        pltpu.SemaphoreType.DMA,
    ],
)
def _sc_spmm(t_hbm, src_hbm, dst_hbm, y_hbm, acc, sidx, didx, rows, sem):
    cid = lax.axis_index("c")
    sid = lax.axis_index("s")

    def fill(i, carry):
        for j in range(D // 16):
            rows[i, pl.ds(j * 16, 16)] = jnp.zeros((16,), jnp.float32)
        return carry
    lax.fori_loop(0, CH, fill, 0)

    for j in range(RPT // CH):
        pltpu.sync_copy(rows, acc.at[pl.ds(sid * RPT + j * CH, CH)])
    plsc.subcore_barrier()

    base = (cid * 16 + sid) * EPT

    def body(c, carry):
        off = base + c * CH
        pltpu.sync_copy(src_hbm.at[pl.ds(off, CH)], sidx)
        pltpu.sync_copy(dst_hbm.at[pl.ds(off, CH)], didx)
        pltpu.async_copy(t_hbm.at[sidx], rows, sem).wait()
        pltpu.sync_copy(rows, acc.at[didx], add=True)
        return carry
    lax.fori_loop(0, CHUNKS, body, 0)
    plsc.subcore_barrier()

    pltpu.sync_copy(acc.at[pl.ds(sid * RPT, RPT)],
                    y_hbm.at[cid, pl.ds(sid * RPT, RPT)])


def _prelu(o):
    return jnp.where(o >= 0.0, o, 0.25 * o)


def _tc_prep_body(degp_ref, x_ref, w_ref, dis_ref, t_ref, acc_ref):
    dp = degp_ref[...]
    deg = dp[0, :, :1] + dp[1, :, :1]
    dis = jnp.where(deg > 0.0, lax.rsqrt(jnp.maximum(deg, 1e-12)), 0.0)
    dis_b = jnp.broadcast_to(dis, (BR, D))
    x = x_ref[...]
    dis_ref[...] = dis_b
    t_ref[...] = dis_b * x
    acc_ref[...] = jnp.dot(x, w_ref[...], preferred_element_type=jnp.float32)


_tc_prep = pl.pallas_call(
    _tc_prep_body,
    grid=(GRID,),
    in_specs=[
        pl.BlockSpec((2, BR, D), lambda i: (0, i, 0)),
        pl.BlockSpec((BR, D), lambda i: (i, 0)),
        pl.BlockSpec((D, D), lambda i: (0, 0)),
    ],
    out_specs=[
        pl.BlockSpec((BR, D), lambda i: (i, 0)),
        pl.BlockSpec((BR, D), lambda i: (i, 0)),
        pl.BlockSpec((BR, D), lambda i: (i, 0)),
    ],
    out_shape=[
        jax.ShapeDtypeStruct((NP, D), jnp.float32),
        jax.ShapeDtypeStruct((NP, D), jnp.float32),
        jax.ShapeDtypeStruct((NP, D), jnp.float32),
    ],
)


def _tc_hop_body(y_ref, dis_ref, acc_ref, w_ref, t_ref, accout_ref):
    y = y_ref[...]
    dis = dis_ref[...]
    xk = dis * (y[0] + y[1])
    accout_ref[...] = acc_ref[...] + jnp.dot(
        xk, w_ref[...], preferred_element_type=jnp.float32)
    t_ref[...] = dis * xk


_tc_hop = pl.pallas_call(
    _tc_hop_body,
    grid=(GRID,),
    in_specs=[
        pl.BlockSpec((2, BR, D), lambda i: (0, i, 0)),
        pl.BlockSpec((BR, D), lambda i: (i, 0)),
        pl.BlockSpec((BR, D), lambda i: (i, 0)),
        pl.BlockSpec((D, D), lambda i: (0, 0)),
    ],
    out_specs=[
        pl.BlockSpec((BR, D), lambda i: (i, 0)),
        pl.BlockSpec((BR, D), lambda i: (i, 0)),
    ],
    out_shape=[
        jax.ShapeDtypeStruct((NP, D), jnp.float32),
        jax.ShapeDtypeStruct((NP, D), jnp.float32),
    ],
)


def _tc_l1_end_body(y_ref, dis_ref, acc_ref, w_ref, b_ref, w20_ref,
                    t_ref, acc2_ref):
    y = y_ref[...]
    dis = dis_ref[...]
    xk = dis * (y[0] + y[1])
    o = acc_ref[...] + jnp.dot(
        xk, w_ref[...], preferred_element_type=jnp.float32) + b_ref[...]
    h = _prelu(o)
    t_ref[...] = dis * h
    acc2_ref[...] = jnp.dot(h, w20_ref[...], preferred_element_type=jnp.float32)


_tc_l1_end = pl.pallas_call(
    _tc_l1_end_body,
    grid=(GRID,),
    in_specs=[
        pl.BlockSpec((2, BR, D), lambda i: (0, i, 0)),
        pl.BlockSpec((BR, D), lambda i: (i, 0)),
        pl.BlockSpec((BR, D), lambda i: (i, 0)),
        pl.BlockSpec((D, D), lambda i: (0, 0)),
        pl.BlockSpec((1, D), lambda i: (0, 0)),
        pl.BlockSpec((D, D), lambda i: (0, 0)),
    ],
    out_specs=[
        pl.BlockSpec((BR, D), lambda i: (i, 0)),
        pl.BlockSpec((BR, D), lambda i: (i, 0)),
    ],
    out_shape=[
        jax.ShapeDtypeStruct((NP, D), jnp.float32),
        jax.ShapeDtypeStruct((NP, D), jnp.float32),
    ],
)


def _tc_l2_end_body(y_ref, dis_ref, acc_ref, w_ref, b_ref, batch_ref,
                    wout_ref, bout_ref, out_ref, pool_ref):
    i = pl.program_id(0)
    y = y_ref[...]
    xk = dis_ref[...] * (y[0] + y[1])
    o = acc_ref[...] + jnp.dot(
        xk, w_ref[...], preferred_element_type=jnp.float32) + b_ref[...]
    h2 = _prelu(o)
    b = batch_ref[0]
    gids = lax.broadcasted_iota(jnp.int32, (G, BR), 0)
    onehot = (gids == b).astype(jnp.float32)
    part = jnp.dot(onehot, h2, preferred_element_type=jnp.float32)

    @pl.when(i == 0)
    def _():
        pool_ref[...] = part

    @pl.when(i > 0)
    def _():
        pool_ref[...] = pool_ref[...] + part

    @pl.when(i == GRID - 1)
    def _():
        out_ref[...] = jnp.dot(
            pool_ref[...], wout_ref[...],
            preferred_element_type=jnp.float32) + bout_ref[...]


_tc_l2_end = pl.pallas_call(
    _tc_l2_end_body,
    grid=(GRID,),
    in_specs=[
        pl.BlockSpec((2, BR, D), lambda i: (0, i, 0)),
        pl.BlockSpec((BR, D), lambda i: (i, 0)),
        pl.BlockSpec((BR, D), lambda i: (i, 0)),
        pl.BlockSpec((D, D), lambda i: (0, 0)),
        pl.BlockSpec((1, D), lambda i: (0, 0)),
        pl.BlockSpec((1, 1, BR), lambda i: (i, 0, 0)),
        pl.BlockSpec((D, D), lambda i: (0, 0)),
        pl.BlockSpec((1, D), lambda i: (0, 0)),
    ],
    out_specs=pl.BlockSpec((G, D), lambda i: (0, 0)),
    out_shape=jax.ShapeDtypeStruct((G, D), jnp.float32),
    scratch_shapes=[pltpu.VMEM((G, D), jnp.float32)],
)


def kernel(x, edge_index, batch, W1, b1, W2, b2, Wout, bout):
    src = edge_index[0]
    dst = edge_index[1]
    pad_e = EP - E
    srcp = jnp.concatenate([src, jnp.zeros((pad_e,), jnp.int32)])
    dstp = jnp.concatenate([dst, jnp.full((pad_e,), N, jnp.int32)])
    xp = jnp.pad(x, ((0, NP - N), (0, 0)))
    batchp = jnp.pad(batch, (0, NP - N), constant_values=G).reshape(GRID, 1, BR)
    b1r = b1.reshape(1, D)
    b2r = b2.reshape(1, D)
    woutp = jnp.pad(Wout, ((0, 0), (0, D - 1)))
    boutp = jnp.pad(bout, (0, D - 1)).reshape(1, D)

    degp = _sc_degree(dstp)
    dis_b, t, acc = _tc_prep(degp, xp, W1[0])
    for k in (1, 2):
        y = _sc_spmm(t, srcp, dstp)
        t, acc = _tc_hop(y, dis_b, acc, W1[k])
    y = _sc_spmm(t, srcp, dstp)
    t, acc = _tc_l1_end(y, dis_b, acc, W1[3], b1r, W2[0])
    for k in (1, 2):
        y = _sc_spmm(t, srcp, dstp)
        t, acc = _tc_hop(y, dis_b, acc, W2[k])
    y = _sc_spmm(t, srcp, dstp)
    out = _tc_l2_end(y, dis_b, acc, W2[3], b2r, batchp, woutp, boutp)
    return out[:, :1]
